# baseline (speedup 1.0000x reference)
import functools

import jax
import jax.numpy as jnp
from jax import lax
from jax.experimental import pallas as pl
from jax.experimental.pallas import tpu as pltpu
from jax.experimental.pallas import tpu_sc as plsc

_LANES = 16


@functools.lru_cache(maxsize=None)
def _build_sc_add(B, S, D, dtype):
    mesh = plsc.VectorSubcoreMesh(core_axis_name="c", subcore_axis_name="s")
    NC, NS = mesh.num_cores, mesh.num_subcores
    NW = NC * NS
    SPW = S // NW
    R = 8
    NT = SPW // R
    CH = D // _LANES

    scratch = (
        [pltpu.VMEM((R, D), dtype) for _ in range(2)]
        + [pltpu.VMEM((B, R, D), dtype) for _ in range(2)]
        + [pltpu.VMEM((B, R, D), dtype) for _ in range(2)]
        + [pltpu.SemaphoreType.DMA for _ in range(4)]
    )

    @functools.partial(
        pl.kernel,
        out_type=jax.ShapeDtypeStruct((B, S, D), dtype),
        mesh=mesh,
        scratch_types=scratch,
    )
    def k(x_hbm, pos_hbm, out_hbm, *scr):
        pbuf = scr[0:2]
        xbuf = scr[2:4]
        obuf = scr[4:6]
        in_sem = scr[6:8]
        out_sem = scr[8:10]

        wid = lax.axis_index("s") * NC + lax.axis_index("c")
        p0 = wid * SPW

        ins, outs = {}, {}

        def issue_in(t):
            sl = t % 2
            row0 = p0 + t * R
            ins[t] = [
                pltpu.async_copy(pos_hbm.at[pl.ds(row0, R)], pbuf[sl], in_sem[sl]),
                pltpu.async_copy(x_hbm.at[:, pl.ds(row0, R)], xbuf[sl], in_sem[sl]),
            ]

        def issue_out(t):
            sl = t % 2
            row0 = p0 + t * R
            outs[t] = [
                pltpu.async_copy(obuf[sl], out_hbm.at[:, pl.ds(row0, R)], out_sem[sl])
            ]

        def compute(t):
            sl = t % 2
            pv = pbuf[sl]
            xb = xbuf[sl]
            ob = obuf[sl]

            @plsc.parallel_loop(0, R * CH, 1, unroll=4)
            def _(i):
                r = i // CH
                col = (i % CH) * _LANES
                p = pv[r, pl.ds(col, _LANES)]
                for b in range(B):
                    ob[b, r, pl.ds(col, _LANES)] = xb[b, r, pl.ds(col, _LANES)] + p

        issue_in(0)
        issue_in(1)
        for t in range(NT):
            for d in ins.pop(t):
                d.wait()
            if t >= 2:
                for d in outs.pop(t - 2):
                    d.wait()
            compute(t)
            issue_out(t)
            if t + 2 < NT:
                issue_in(t + 2)
        for t in sorted(outs):
            for d in outs[t]:
                d.wait()

    return k


def kernel(x, pos_table):
    B, S, D = x.shape
    return _build_sc_add(B, S, D, x.dtype)(x, pos_table)
